# baseline (speedup 1.0000x reference)
import jax
import jax.numpy as jnp
from jax import lax
from jax.experimental import pallas as pl
from jax.experimental.pallas import tpu as pltpu
from jax.experimental.pallas import tpu_sc as plsc

B = 16384
D = 64
NNEG = 20
NC = 2
NS = 16
LANES = 16
NW = NC * NS
PW = B // NW
GB = LANES
NG = PW // GB
GROWS = GB * NNEG
IDX_CHUNK = 128


NBUF = 4


def _sc_scores_body(u_emb, v_emb, pos_u, pos_v, neg_flat, out,
                    posu_v, posv_v, negi_v, scores_v,
                    ubufs, vbufs, nbufs, sems):
    wid = lax.axis_index("s") * NC + lax.axis_index("c")
    base = wid * PW

    pltpu.sync_copy(pos_u.at[pl.ds(base, PW)], posu_v)
    pltpu.sync_copy(pos_v.at[pl.ds(base, PW)], posv_v)
    pltpu.sync_copy(neg_flat.at[pl.ds(base * NNEG, PW * NNEG)], negi_v)

    def transfers(g, k):
        off = g * GROWS
        gsl = pl.ds(g * GB, GB)
        yield u_emb.at[posu_v.at[gsl]], ubufs[k], sems[k]
        yield v_emb.at[posv_v.at[gsl]], vbufs[k], sems[k]
        yield (v_emb.at[negi_v.at[pl.ds(off, IDX_CHUNK)]],
               nbufs[k].at[pl.ds(0, IDX_CHUNK)], sems[k])
        yield (v_emb.at[negi_v.at[pl.ds(off + IDX_CHUNK, IDX_CHUNK)]],
               nbufs[k].at[pl.ds(IDX_CHUNK, IDX_CHUNK)], sems[k])
        yield (v_emb.at[negi_v.at[pl.ds(off + 2 * IDX_CHUNK, GROWS - 2 * IDX_CHUNK)]],
               nbufs[k].at[pl.ds(2 * IDX_CHUNK, GROWS - 2 * IDX_CHUNK)], sems[k])

    def issue(g, k):
        for src, dst, sem in transfers(g, k):
            pltpu.async_copy(src, dst, sem)

    def drain(g, k):
        for src, dst, sem in transfers(g, k):
            pltpu.make_async_copy(src, dst, sem).wait()

    liota = lax.iota(jnp.int32, LANES)
    l20 = liota * NNEG

    def compute(g, k):
        ubuf, vbuf, buf = ubufs[k], vbufs[k], nbufs[k]

        def jbody(j, carry):
            accp, accn = carry
            colj = jnp.bitwise_and(liota + j, D - 1)
            ucol = plsc.load_gather(ubuf, [liota, colj])
            vcol = plsc.load_gather(vbuf, [liota, colj])
            accp = accp + ucol * vcol
            accn = tuple(
                accn[n] + ucol * plsc.load_gather(buf, [l20 + n, colj])
                for n in range(NNEG))
            return accp, accn

        zero = jnp.zeros((LANES,), jnp.float32)
        accp, accn = lax.fori_loop(0, D, jbody, (zero, (zero,) * NNEG))
        sl = pl.ds(g * GB, GB)
        for n in range(NNEG):
            scores_v[n, sl] = accn[n]
        scores_v[NNEG, sl] = accp

    for k in range(NBUF):
        issue(k, k)

    def outer(q, carry):
        for k in range(NBUF):
            g = q * NBUF + k
            drain(g, k)
            compute(g, k)

            @pl.when(g + NBUF < NG)
            def _():
                issue(g + NBUF, k)

        return carry

    lax.fori_loop(0, NG // NBUF, outer, 0)
    pltpu.sync_copy(scores_v, out.at[wid])


def _tc_reduce_body(scores_ref, out_ref):
    x = jnp.clip(scores_ref[...], -10.0, 10.0)
    n = lax.broadcasted_iota(jnp.int32, x.shape, 1)
    t = jnp.where(n == NNEG, -x, x)
    val = jnp.maximum(t, 0.0) + jnp.log1p(jnp.exp(-jnp.abs(x)))
    out_ref[0, 0] = jnp.sum(val) / B


def kernel(pos_u, pos_v, neg_v, u_emb, frozen_emb, v_emb):
    del frozen_emb
    neg_flat = neg_v.reshape(-1)
    mesh = plsc.VectorSubcoreMesh(core_axis_name="c", subcore_axis_name="s",
                                  num_cores=NC, num_subcores=NS)
    scores = pl.kernel(
        _sc_scores_body,
        out_type=jax.ShapeDtypeStruct((NW, NNEG + 1, PW), jnp.float32),
        mesh=mesh,
        compiler_params=pltpu.CompilerParams(needs_layout_passes=False,
                                             use_tc_tiling_on_sc=False,
                                             disable_bounds_checks=True,
                                             disable_semaphore_checks=True,
                                             skip_device_barrier=True),
        scratch_types=[
            pltpu.VMEM((PW,), jnp.int32),
            pltpu.VMEM((PW,), jnp.int32),
            pltpu.VMEM((PW * NNEG,), jnp.int32),
            pltpu.VMEM((NNEG + 1, PW), jnp.float32),
            [pltpu.VMEM((GB, D), jnp.float32) for _ in range(NBUF)],
            [pltpu.VMEM((GB, D), jnp.float32) for _ in range(NBUF)],
            [pltpu.VMEM((GROWS, D), jnp.float32) for _ in range(NBUF)],
            [pltpu.SemaphoreType.DMA for _ in range(NBUF)],
        ],
    )(u_emb, v_emb, pos_u, pos_v, neg_flat)
    loss = pl.pallas_call(
        _tc_reduce_body,
        out_shape=jax.ShapeDtypeStruct((1, 1), jnp.float32),
        out_specs=pl.BlockSpec(memory_space=pltpu.SMEM),
    )(scores)
    return loss[0, 0]


# speedup vs baseline: 1.0354x; 1.0354x over previous
import jax
import jax.numpy as jnp
from jax import lax
from jax.experimental import pallas as pl
from jax.experimental.pallas import tpu as pltpu
from jax.experimental.pallas import tpu_sc as plsc

B = 16384
D = 64
NNEG = 20
NC = 2
NS = 16
LANES = 16
NW = NC * NS
PW = B // NW
GB = LANES
NG = PW // GB
GROWS = GB * NNEG
IDX_CHUNK = 128


def _sc_scores_body(u_emb, v_emb, pos_u, pos_v, neg_flat, out,
                    posu_v, posv_v, negi_v, u_rows, v_rows, nbuf0, nbuf1,
                    scores_v, sem_uv, sem_n0, sem_n1):
    wid = lax.axis_index("s") * NC + lax.axis_index("c")
    base = wid * PW

    pltpu.sync_copy(pos_u.at[pl.ds(base, PW)], posu_v)
    pltpu.sync_copy(pos_v.at[pl.ds(base, PW)], posv_v)
    pltpu.sync_copy(neg_flat.at[pl.ds(base * NNEG, PW * NNEG)], negi_v)

    for c in range(PW // IDX_CHUNK):
        s = pl.ds(c * IDX_CHUNK, IDX_CHUNK)
        pltpu.async_copy(u_emb.at[posu_v.at[s]], u_rows.at[s], sem_uv)
        pltpu.async_copy(v_emb.at[posv_v.at[s]], v_rows.at[s], sem_uv)
    for c in range(PW // IDX_CHUNK):
        s = pl.ds(c * IDX_CHUNK, IDX_CHUNK)
        pltpu.make_async_copy(u_emb.at[posu_v.at[s]], u_rows.at[s], sem_uv).wait()
        pltpu.make_async_copy(v_emb.at[posv_v.at[s]], v_rows.at[s], sem_uv).wait()

    def issue(g, buf, sem):
        off = g * GROWS
        pltpu.async_copy(v_emb.at[negi_v.at[pl.ds(off, IDX_CHUNK)]],
                         buf.at[pl.ds(0, IDX_CHUNK)], sem)
        pltpu.async_copy(v_emb.at[negi_v.at[pl.ds(off + IDX_CHUNK, IDX_CHUNK)]],
                         buf.at[pl.ds(IDX_CHUNK, IDX_CHUNK)], sem)
        pltpu.async_copy(v_emb.at[negi_v.at[pl.ds(off + 2 * IDX_CHUNK, GROWS - 2 * IDX_CHUNK)]],
                         buf.at[pl.ds(2 * IDX_CHUNK, GROWS - 2 * IDX_CHUNK)], sem)

    def drain(g, buf, sem):
        off = g * GROWS
        pltpu.make_async_copy(v_emb.at[negi_v.at[pl.ds(off, IDX_CHUNK)]],
                              buf.at[pl.ds(0, IDX_CHUNK)], sem).wait()
        pltpu.make_async_copy(v_emb.at[negi_v.at[pl.ds(off + IDX_CHUNK, IDX_CHUNK)]],
                              buf.at[pl.ds(IDX_CHUNK, IDX_CHUNK)], sem).wait()
        pltpu.make_async_copy(v_emb.at[negi_v.at[pl.ds(off + 2 * IDX_CHUNK, GROWS - 2 * IDX_CHUNK)]],
                              buf.at[pl.ds(2 * IDX_CHUNK, GROWS - 2 * IDX_CHUNK)], sem).wait()

    liota = lax.iota(jnp.int32, LANES)
    l20 = liota * NNEG

    def compute(g, buf):
        rrow = liota + g * GB

        def jbody(j, carry):
            accp, accn = carry
            colj = jnp.bitwise_and(liota + j, D - 1)
            ucol = plsc.load_gather(u_rows, [rrow, colj])
            vcol = plsc.load_gather(v_rows, [rrow, colj])
            accp = accp + ucol * vcol
            accn = tuple(
                accn[n] + ucol * plsc.load_gather(buf, [l20 + n, colj])
                for n in range(NNEG))
            return accp, accn

        zero = jnp.zeros((LANES,), jnp.float32)
        accp, accn = lax.fori_loop(0, D, jbody, (zero, (zero,) * NNEG))
        sl = pl.ds(g * GB, GB)
        for n in range(NNEG):
            scores_v[n, sl] = accn[n]
        scores_v[NNEG, sl] = accp

    issue(0, nbuf0, sem_n0)

    def outer(gg, carry):
        g0 = gg * 2
        issue(g0 + 1, nbuf1, sem_n1)
        drain(g0, nbuf0, sem_n0)
        compute(g0, nbuf0)

        @pl.when(g0 + 2 < NG)
        def _():
            issue(g0 + 2, nbuf0, sem_n0)

        drain(g0 + 1, nbuf1, sem_n1)
        compute(g0 + 1, nbuf1)
        return carry

    lax.fori_loop(0, NG // 2, outer, 0)
    pltpu.sync_copy(scores_v, out.at[wid])


def _tc_reduce_body(scores_ref, out_ref):
    x = jnp.clip(scores_ref[...], -10.0, 10.0)
    n = lax.broadcasted_iota(jnp.int32, x.shape, 1)
    t = jnp.where(n == NNEG, -x, x)
    val = jnp.maximum(t, 0.0) + jnp.log1p(jnp.exp(-jnp.abs(x)))
    out_ref[0, 0] = jnp.sum(val) / B


def kernel(pos_u, pos_v, neg_v, u_emb, frozen_emb, v_emb):
    del frozen_emb
    neg_flat = neg_v.reshape(-1)
    mesh = plsc.VectorSubcoreMesh(core_axis_name="c", subcore_axis_name="s",
                                  num_cores=NC, num_subcores=NS)
    scores = pl.kernel(
        _sc_scores_body,
        out_type=jax.ShapeDtypeStruct((NW, NNEG + 1, PW), jnp.float32),
        mesh=mesh,
        compiler_params=pltpu.CompilerParams(needs_layout_passes=False,
                                             use_tc_tiling_on_sc=False,
                                             disable_bounds_checks=True,
                                             disable_semaphore_checks=True,
                                             skip_device_barrier=True),
        scratch_types=[
            pltpu.VMEM((PW,), jnp.int32),
            pltpu.VMEM((PW,), jnp.int32),
            pltpu.VMEM((PW * NNEG,), jnp.int32),
            pltpu.VMEM((PW, D), jnp.float32),
            pltpu.VMEM((PW, D), jnp.float32),
            pltpu.VMEM((GROWS, D), jnp.float32),
            pltpu.VMEM((GROWS, D), jnp.float32),
            pltpu.VMEM((NNEG + 1, PW), jnp.float32),
            pltpu.SemaphoreType.DMA,
            pltpu.SemaphoreType.DMA,
            pltpu.SemaphoreType.DMA,
        ],
    )(u_emb, v_emb, pos_u, pos_v, neg_flat)
    loss = pl.pallas_call(
        _tc_reduce_body,
        out_shape=jax.ShapeDtypeStruct((1, 1), jnp.float32),
        out_specs=pl.BlockSpec(memory_space=pltpu.SMEM),
    )(scores)
    return loss[0, 0]
